# baseline (speedup 1.0000x reference)
import functools
import jax
import jax.numpy as jnp
from jax import lax
from jax.experimental import pallas as pl
from jax.experimental.pallas import tpu as pltpu
from jax.experimental.pallas import tpu_sc as plsc

_K = 64
_NC, _NS, _L = 2, 16, 16
_NW = _NC * _NS
_B, _N = 64, 8192
_RPW = _B // _NW
_NV = _N // _L
_NBKT = 256
_INT_MIN = -2147483648
_MASK31 = 0x7FFFFFFF


def _scalar(x):
    return x if x.ndim == 0 else jnp.max(x)


def _key_of(v):
    b = lax.bitcast_convert_type(v, jnp.int32)
    key = jnp.where(b >= 0, b, b ^ jnp.int32(_MASK31))
    return jnp.where(b == jnp.int32(_INT_MIN), jnp.int32(0), key)


def _sc_body(x_hbm, w_hbm, st_hbm,
             rows_v, ws_v, key_v, hist_v, totals_v, ck_v, ci_v, st_v,
             sin0, sin1, sout0, sout1):
    wid = lax.axis_index("c") * _NS + lax.axis_index("s")
    iota = lax.iota(jnp.int32, _L)
    onesi = jnp.ones((_L,), jnp.int32)
    zeroi = jnp.zeros((_L,), jnp.int32)
    inv_k = jnp.float32(1.0 / _K)

    row0 = wid * _RPW
    hin = [pltpu.async_copy(x_hbm.at[row0 + rr], rows_v.at[rr], sem)
           for rr, sem in enumerate((sin0, sin1))]
    houts = []

    st_acc = jnp.zeros((_L,), jnp.float32)

    @plsc.parallel_loop(0, _NBKT, unroll=8)
    def _(j):
        hist_v[pl.ds(j * _L, _L)] = zeroi

    for rr in range(_RPW):
        row_v = rows_v.at[rr]
        w_v = ws_v.at[rr]
        hin[rr].wait()

        @plsc.parallel_loop(0, _NV, unroll=8)
        def _(j):
            key = _key_of(row_v[pl.ds(j * _L, _L)])
            key_v[pl.ds(j * _L, _L)] = key
            bkt = (key >> 24) + 128
            plsc.addupdate_scatter(hist_v, [(iota << 8) + bkt], onesi)

        def tbody(g, gtot):
            acc = zeroi
            for l in range(_L):
                acc = acc + hist_v[pl.ds(l * _NBKT + g * _L, _L)]
                hist_v[pl.ds(l * _NBKT + g * _L, _L)] = zeroi
            totals_v[pl.ds(g * _L, _L)] = acc
            return jnp.where(iota == g, jnp.broadcast_to(jnp.sum(acc), (_L,)),
                             gtot)
        gtot = lax.fori_loop(0, _NBKT // _L, tbody, zeroi)

        cs = plsc.cumsum(lax.rev(gtot, (0,)))
        i0 = _scalar(plsc.all_reduce_ffs(cs >= _K))
        em = iota == i0
        above = (jnp.max(jnp.where(em, cs, zeroi))
                 - jnp.max(jnp.where(em, lax.rev(gtot, (0,)), zeroi)))
        grp = 15 - i0
        trev = lax.rev(totals_v[pl.ds(grp * _L, _L)], (0,))
        cs2 = plsc.cumsum(trev) + above
        j0 = _scalar(plsc.all_reduce_ffs(cs2 >= _K))
        em2 = iota == j0
        b1 = (grp << 4) + 15 - j0
        cgt1 = (jnp.max(jnp.where(em2, cs2, zeroi))
                - jnp.max(jnp.where(em2, trev, zeroi)))

        top1 = b1 - 128
        @plsc.parallel_loop(0, _NV, unroll=8, carry=zeroi)
        def cntv(j, cnt):
            key = key_v[pl.ds(j * _L, _L)]
            m = (key >> 24) == top1
            mv = jnp.where(m, onesi, zeroi)
            pos = cnt + plsc.cumsum(mv) - mv
            col = (j << 4) + iota
            plsc.store_scatter(ck_v, [pos], key, mask=m)
            plsc.store_scatter(ci_v, [pos], col, mask=m)
            return cnt + plsc.all_reduce_population_count(m)
        n2 = jnp.max(cntv)
        nv2 = (n2 + _L - 1) >> 4

        base1 = (b1 - 128) << 24
        def bitstep(s, tk):
            cand = tk | (jnp.int32(1) << (23 - s))
            @plsc.parallel_loop(0, nv2, unroll=4, carry=zeroi)
            def c(vi, c):
                kk = ck_v[pl.ds(vi * _L, _L)]
                valid = ((vi << 4) + iota) < n2
                return c + plsc.all_reduce_population_count(
                    valid & (kk >= cand))
            return jnp.where(cgt1 + jnp.max(c) >= _K, cand, tk)
        tkey = lax.fori_loop(0, 24, bitstep, base1)

        @plsc.parallel_loop(0, nv2, unroll=4, carry=zeroi)
        def cgtv(vi, c):
            kk = ck_v[pl.ds(vi * _L, _L)]
            valid = ((vi << 4) + iota) < n2
            return c + plsc.all_reduce_population_count(valid & (kk > tkey))
        cgtS = cgt1 + jnp.max(cgtv)
        r = _K - cgtS

        vmax0 = jnp.full((_L,), _INT_MIN, jnp.int32)
        sacc0 = jnp.zeros((_L,), jnp.float32)
        @plsc.parallel_loop(0, _NV, unroll=8, carry=(sacc0, vmax0, zeroi))
        def p3out(j, carry):
            sacc, vmax, vidx = carry
            v = row_v[pl.ds(j * _L, _L)]
            key = key_v[pl.ds(j * _L, _L)]
            g = key > tkey
            w_v[pl.ds(j * _L, _L)] = jnp.where(g, inv_k, jnp.float32(0.0))
            m = key > vmax
            col = (j << 4) + iota
            return (sacc + jnp.where(g, v, jnp.float32(0.0)),
                    jnp.where(m, key, vmax), jnp.where(m, col, vidx))
        saccv, vmax, vidx = p3out
        sum_gt = jnp.sum(saccv)
        mx = jnp.max(vmax)
        bi_row = jnp.min(jnp.where(vmax == mx, vidx, jnp.int32(_N)))

        wts = jnp.full((_L,), 1.0 / _K, jnp.float32)
        def tie(vi, c):
            kk = ck_v[pl.ds(vi * _L, _L)]
            ii = ci_v[pl.ds(vi * _L, _L)]
            valid = ((vi << 4) + iota) < n2
            e = valid & (kk == tkey)
            ev = jnp.where(e, onesi, zeroi)
            rank = c + plsc.cumsum(ev) - ev
            plsc.store_scatter(w_v, [ii], wts, mask=e & (rank < r))
            return c + plsc.all_reduce_population_count(e)
        lax.fori_loop(0, nv2, tie, zeroi)

        houts.append(pltpu.async_copy(
            w_v, w_hbm.at[row0 + rr], (sout0, sout1)[rr]))

        tbits = jnp.where(tkey >= 0, tkey, tkey ^ jnp.int32(_MASK31))
        tval = lax.bitcast_convert_type(jnp.broadcast_to(tbits, (_L,)),
                                        jnp.float32)
        rv = jnp.broadcast_to(r, (_L,)).astype(jnp.float32)
        fsv = (jnp.broadcast_to(sum_gt, (_L,)) + rv * tval) * inv_k
        biv = lax.bitcast_convert_type(jnp.broadcast_to(bi_row, (_L,)),
                                       jnp.float32)
        st_acc = jnp.where(iota == rr, fsv, st_acc)
        st_acc = jnp.where(iota == _RPW + rr, biv, st_acc)

    st_v[pl.ds(0, _L)] = st_acc
    pltpu.sync_copy(st_v, st_hbm.at[wid])
    for h in houts:
        h.wait()


@jax.jit
def kernel(chunk_scores):
    B, N = chunk_scores.shape
    mesh = plsc.VectorSubcoreMesh(core_axis_name="c", subcore_axis_name="s")
    w, st = pl.kernel(
        _sc_body,
        out_type=(
            jax.ShapeDtypeStruct((B, N), jnp.float32),
            jax.ShapeDtypeStruct((_NW, _L), jnp.float32),
        ),
        mesh=mesh,
        compiler_params=pltpu.CompilerParams(needs_layout_passes=False,
                                             use_tc_tiling_on_sc=False,
                                             disable_bounds_checks=True,
                                             skip_device_barrier=True),
        scratch_types=[
            pltpu.VMEM((_RPW, _N), jnp.float32),
            pltpu.VMEM((_RPW, _N), jnp.float32),
            pltpu.VMEM((_N,), jnp.int32),
            pltpu.VMEM((_NBKT * _L,), jnp.int32),
            pltpu.VMEM((_NBKT,), jnp.int32),
            pltpu.VMEM((_N + _L,), jnp.int32),
            pltpu.VMEM((_N + _L,), jnp.int32),
            pltpu.VMEM((_L,), jnp.float32),
            pltpu.SemaphoreType.DMA,
            pltpu.SemaphoreType.DMA,
            pltpu.SemaphoreType.DMA,
            pltpu.SemaphoreType.DMA,
        ],
    )(chunk_scores)
    final_score = st[:, :_RPW].reshape(B)
    best_idx = jax.lax.bitcast_convert_type(
        st[:, _RPW:2 * _RPW].reshape(B), jnp.int32)
    return final_score, best_idx, w


# speedup vs baseline: 1.0510x; 1.0510x over previous
import jax
import jax.numpy as jnp
from jax import lax
from jax.experimental import pallas as pl
from jax.experimental.pallas import tpu as pltpu
from jax.experimental.pallas import tpu_sc as plsc

_K = 64
_NC, _NS, _L = 2, 16, 16
_NW = _NC * _NS
_B, _N = 64, 8192
_RPW = _B // _NW
_NV = _N // _L
_NBKT = 256
_INT_MIN = -2147483648
_MASK31 = 0x7FFFFFFF


def _scalar(x):
    return x if x.ndim == 0 else jnp.max(x)


def _key_of(v):
    b = lax.bitcast_convert_type(v, jnp.int32)
    key = jnp.where(b >= 0, b, b ^ jnp.int32(_MASK31))
    return jnp.where(b == jnp.int32(_INT_MIN), jnp.int32(0), key)


def _sc_body(x_hbm, w_hbm, st_hbm,
             rows_v, ws_v, keys_v, hist_v, totals_v, ck_v, ci_v, st_v,
             sin0, sin1, sout0, sout1):
    wid = lax.axis_index("c") * _NS + lax.axis_index("s")
    iota = lax.iota(jnp.int32, _L)
    onesi = jnp.ones((_L,), jnp.int32)
    zeroi = jnp.zeros((_L,), jnp.int32)
    inv_k = jnp.float32(1.0 / _K)
    R = range(_RPW)

    row0 = wid * _RPW
    hin = [pltpu.async_copy(x_hbm.at[row0 + rr], rows_v.at[rr], sem)
           for rr, sem in enumerate((sin0, sin1))]

    @plsc.parallel_loop(0, _RPW * _NBKT, unroll=8)
    def _(j):
        hist_v[pl.ds(j * _L, _L)] = zeroi

    for h in hin:
        h.wait()

    @plsc.parallel_loop(0, _NV, unroll=4)
    def _(j):
        for rr in R:
            key = _key_of(rows_v.at[rr][pl.ds(j * _L, _L)])
            keys_v.at[rr][pl.ds(j * _L, _L)] = key
            bkt = (key >> 24) + 128
            plsc.addupdate_scatter(
                hist_v, [(rr * _NBKT * _L) + (iota << 8) + bkt], onesi)

    def tbody(g, gts):
        out = []
        for rr, gtot in zip(R, gts):
            acc = zeroi
            for l in range(_L):
                acc = acc + hist_v[pl.ds((rr * _L + l) * _NBKT + g * _L, _L)]
            totals_v.at[rr][pl.ds(g * _L, _L)] = acc
            out.append(jnp.where(iota == g,
                                 jnp.broadcast_to(jnp.sum(acc), (_L,)), gtot))
        return tuple(out)
    gtots = lax.fori_loop(0, _NBKT // _L, tbody, (zeroi,) * _RPW)

    b1s, cgt1s = [], []
    for rr in R:
        gtot = gtots[rr]
        cs = plsc.cumsum(lax.rev(gtot, (0,)))
        i0 = _scalar(plsc.all_reduce_ffs(cs >= _K))
        em = iota == i0
        above = (jnp.max(jnp.where(em, cs, zeroi))
                 - jnp.max(jnp.where(em, lax.rev(gtot, (0,)), zeroi)))
        grp = 15 - i0
        trev = lax.rev(totals_v.at[rr][pl.ds(grp * _L, _L)], (0,))
        cs2 = plsc.cumsum(trev) + above
        j0 = _scalar(plsc.all_reduce_ffs(cs2 >= _K))
        em2 = iota == j0
        b1s.append((grp << 4) + 15 - j0)
        cgt1s.append(jnp.max(jnp.where(em2, cs2, zeroi))
                     - jnp.max(jnp.where(em2, trev, zeroi)))

    top1s = [b1s[rr] - 128 for rr in R]
    @plsc.parallel_loop(0, _NV, unroll=4, carry=(zeroi,) * _RPW)
    def cnts(j, cnt):
        col = (j << 4) + iota
        out = []
        for rr in R:
            key = keys_v.at[rr][pl.ds(j * _L, _L)]
            m = (key >> 24) == top1s[rr]
            mv = jnp.where(m, onesi, zeroi)
            pos = cnt[rr] + plsc.cumsum(mv) - mv
            plsc.store_scatter(ck_v.at[rr], [pos], key, mask=m)
            plsc.store_scatter(ci_v.at[rr], [pos], col, mask=m)
            out.append(cnt[rr] + plsc.all_reduce_population_count(m))
        return tuple(out)
    n2s = [jnp.max(cnts[rr]) for rr in R]
    nv2s = [(n2s[rr] + _L - 1) >> 4 for rr in R]
    nv2m = jnp.maximum(nv2s[0], nv2s[1])

    def bitstep(s, tks):
        cands = [tks[rr] | (jnp.int32(1) << (23 - s)) for rr in R]
        @plsc.parallel_loop(0, nv2m, unroll=2, carry=(zeroi,) * _RPW)
        def cc(vi, c):
            pos = (vi << 4) + iota
            out = []
            for rr in R:
                kk = ck_v.at[rr][pl.ds(vi * _L, _L)]
                valid = pos < n2s[rr]
                out.append(c[rr] + plsc.all_reduce_population_count(
                    valid & (kk >= cands[rr])))
            return tuple(out)
        return tuple(
            jnp.where(cgt1s[rr] + jnp.max(cc[rr]) >= _K, cands[rr], tks[rr])
            for rr in R)
    tkeys = lax.fori_loop(0, 24, bitstep,
                          tuple((b1s[rr] - 128) << 24 for rr in R))

    @plsc.parallel_loop(0, nv2m, unroll=2, carry=(zeroi,) * _RPW)
    def cgts(vi, c):
        pos = (vi << 4) + iota
        out = []
        for rr in R:
            kk = ck_v.at[rr][pl.ds(vi * _L, _L)]
            valid = pos < n2s[rr]
            out.append(c[rr] + plsc.all_reduce_population_count(
                valid & (kk > tkeys[rr])))
        return tuple(out)
    rs = [_K - (cgt1s[rr] + jnp.max(cgts[rr])) for rr in R]

    vmax0 = jnp.full((_L,), _INT_MIN, jnp.int32)
    sacc0 = jnp.zeros((_L,), jnp.float32)
    @plsc.parallel_loop(0, _NV, unroll=4,
                        carry=((sacc0, vmax0, zeroi),) * _RPW)
    def p3out(j, carry):
        col = (j << 4) + iota
        out = []
        for rr in R:
            sacc, vmax, vidx = carry[rr]
            v = rows_v.at[rr][pl.ds(j * _L, _L)]
            key = keys_v.at[rr][pl.ds(j * _L, _L)]
            g = key > tkeys[rr]
            ws_v.at[rr][pl.ds(j * _L, _L)] = jnp.where(g, inv_k,
                                                       jnp.float32(0.0))
            m = key > vmax
            out.append((sacc + jnp.where(g, v, jnp.float32(0.0)),
                        jnp.where(m, key, vmax), jnp.where(m, col, vidx)))
        return tuple(out)

    wts = jnp.full((_L,), 1.0 / _K, jnp.float32)
    @plsc.parallel_loop(0, nv2m, unroll=2, carry=(zeroi,) * _RPW)
    def _(vi, c):
        pos = (vi << 4) + iota
        out = []
        for rr in R:
            kk = ck_v.at[rr][pl.ds(vi * _L, _L)]
            ii = ci_v.at[rr][pl.ds(vi * _L, _L)]
            e = (pos < n2s[rr]) & (kk == tkeys[rr])
            ev = jnp.where(e, onesi, zeroi)
            rank = c[rr] + plsc.cumsum(ev) - ev
            plsc.store_scatter(ws_v.at[rr], [ii], wts,
                               mask=e & (rank < rs[rr]))
            out.append(c[rr] + plsc.all_reduce_population_count(e))
        return tuple(out)

    houts = [pltpu.async_copy(ws_v.at[rr], w_hbm.at[row0 + rr], sem)
             for rr, sem in enumerate((sout0, sout1))]

    st_acc = jnp.zeros((_L,), jnp.float32)
    for rr in R:
        saccv, vmax, vidx = p3out[rr]
        sum_gt = jnp.sum(saccv)
        mx = jnp.max(vmax)
        bi_row = jnp.min(jnp.where(vmax == mx, vidx, jnp.int32(_N)))
        tkey = tkeys[rr]
        tbits = jnp.where(tkey >= 0, tkey, tkey ^ jnp.int32(_MASK31))
        tval = lax.bitcast_convert_type(jnp.broadcast_to(tbits, (_L,)),
                                        jnp.float32)
        rv = jnp.broadcast_to(rs[rr], (_L,)).astype(jnp.float32)
        fsv = (jnp.broadcast_to(sum_gt, (_L,)) + rv * tval) * inv_k
        biv = lax.bitcast_convert_type(jnp.broadcast_to(bi_row, (_L,)),
                                       jnp.float32)
        st_acc = jnp.where(iota == rr, fsv, st_acc)
        st_acc = jnp.where(iota == _RPW + rr, biv, st_acc)

    st_v[pl.ds(0, _L)] = st_acc
    pltpu.sync_copy(st_v, st_hbm.at[wid])
    for h in houts:
        h.wait()


@jax.jit
def kernel(chunk_scores):
    B, N = chunk_scores.shape
    mesh = plsc.VectorSubcoreMesh(core_axis_name="c", subcore_axis_name="s")
    w, st = pl.kernel(
        _sc_body,
        out_type=(
            jax.ShapeDtypeStruct((B, N), jnp.float32),
            jax.ShapeDtypeStruct((_NW, _L), jnp.float32),
        ),
        mesh=mesh,
        compiler_params=pltpu.CompilerParams(needs_layout_passes=False,
                                             use_tc_tiling_on_sc=False,
                                             disable_bounds_checks=True),
        scratch_types=[
            pltpu.VMEM((_RPW, _N), jnp.float32),
            pltpu.VMEM((_RPW, _N), jnp.float32),
            pltpu.VMEM((_RPW, _N), jnp.int32),
            pltpu.VMEM((_RPW * _NBKT * _L,), jnp.int32),
            pltpu.VMEM((_RPW, _NBKT), jnp.int32),
            pltpu.VMEM((_RPW, _N + _L), jnp.int32),
            pltpu.VMEM((_RPW, _N + _L), jnp.int32),
            pltpu.VMEM((_L,), jnp.float32),
            pltpu.SemaphoreType.DMA,
            pltpu.SemaphoreType.DMA,
            pltpu.SemaphoreType.DMA,
            pltpu.SemaphoreType.DMA,
        ],
    )(chunk_scores)
    final_score = st[:, :_RPW].reshape(B)
    best_idx = jax.lax.bitcast_convert_type(
        st[:, _RPW:2 * _RPW].reshape(B), jnp.int32)
    return final_score, best_idx, w
